# Initial kernel scaffold; baseline (speedup 1.0000x reference)
import jax
import jax.numpy as jnp
from jax.experimental import pallas as pl


def kernel(context, cu_seqlens, context_theta):
    raise NotImplementedError("write your pallas kernel here")


# speedup vs baseline: 1.8015x; 1.8015x over previous
import jax
import jax.numpy as jnp
from jax import lax
from jax.experimental import pallas as pl
from jax.experimental.pallas import tpu as pltpu, tpu_sc as plsc

T = 4096
D = 128
LANES = 16
NC = 2
NS = 16
NW = NC * NS
CHUNK = T // NW
CHUNK_E = CHUNK * D
KD = D // LANES

_mesh = plsc.VectorSubcoreMesh(core_axis_name="c", subcore_axis_name="s")
_cparams = pltpu.CompilerParams(needs_layout_passes=False)


def _cu_scalars(cu_vec):
    lane = lax.iota(jnp.int32, LANES)
    cu_f = cu_vec.astype(jnp.float32)
    c1 = jnp.sum(jnp.where(lane == 1, cu_f, 0.0)).astype(jnp.int32)
    c2 = jnp.sum(jnp.where(lane == 2, cu_f, 0.0)).astype(jnp.int32)
    c3 = jnp.sum(jnp.where(lane == 3, cu_f, 0.0)).astype(jnp.int32)
    return c1, c2, c3


def _seg_of(p, c1, c2, c3):
    return ((p >= c1).astype(jnp.int32) + (p >= c2).astype(jnp.int32)
            + (p >= c3).astype(jnp.int32))


def _phase1_body(ctx_hbm, cu_hbm, th_hbm, tnum_hbm, tden_hbm,
                 ctx_v, th_v, cu_v, tn_v, td_v):
    c = lax.axis_index("c")
    s = lax.axis_index("s")
    wid = s * NC + c
    pltpu.sync_copy(ctx_hbm.at[pl.ds(wid * CHUNK_E, CHUNK_E)], ctx_v)
    pltpu.sync_copy(th_hbm, th_v)
    pltpu.sync_copy(cu_hbm, cu_v)
    c1, c2, c3 = _cu_scalars(cu_v[:])
    th = [th_v[pl.ds(LANES * k, LANES)] for k in range(KD)]
    row0 = wid * CHUNK
    zero = jnp.zeros((LANES,), jnp.float32)

    def body(r, carry):
        den = carry[0]
        nums = carry[1:]
        off = r * D
        cks = [ctx_v[pl.ds(off + LANES * k, LANES)] for k in range(KD)]
        acc = cks[0] * th[0]
        for k in range(1, KD):
            acc = acc + cks[k] * th[k]
        e = jnp.exp(jnp.full((LANES,), jnp.sum(acc), jnp.float32))
        rg = row0 + r
        is_start = (rg == c1) | (rg == c2) | (rg == c3)
        kv = jnp.full((LANES,), jnp.where(is_start, 0.0, 1.0), jnp.float32)
        den = den * kv + e
        nums = tuple(n * kv + e * ck for n, ck in zip(nums, cks))
        return (den,) + nums

    res = lax.fori_loop(0, CHUNK, body, (zero,) * (KD + 1))
    td_v[:] = res[0]
    for k in range(KD):
        tn_v[pl.ds(LANES * k, LANES)] = res[1 + k]
    pltpu.sync_copy(tn_v, tnum_hbm.at[pl.ds(wid * D, D)])
    pltpu.sync_copy(td_v, tden_hbm.at[pl.ds(wid * LANES, LANES)])


def _phase2_body(ctx_hbm, cu_hbm, th_hbm, tnum_hbm, tden_hbm, out_hbm,
                 ctx_v, out_v, th_v, cu_v, tn_v, td_v):
    c = lax.axis_index("c")
    s = lax.axis_index("s")
    wid = s * NC + c
    pltpu.sync_copy(ctx_hbm.at[pl.ds(wid * CHUNK_E, CHUNK_E)], ctx_v)
    pltpu.sync_copy(th_hbm, th_v)
    pltpu.sync_copy(cu_hbm, cu_v)
    pltpu.sync_copy(tnum_hbm, tn_v)
    pltpu.sync_copy(tden_hbm, td_v)
    c1, c2, c3 = _cu_scalars(cu_v[:])
    th = [th_v[pl.ds(LANES * k, LANES)] for k in range(KD)]
    row0 = wid * CHUNK
    zero = jnp.zeros((LANES,), jnp.float32)

    s0 = _seg_of(row0, c1, c2, c3)
    cden = zero
    cnum = [zero] * KD
    for wp in range(NW - 1):
        segl = _seg_of(wp * CHUNK + CHUNK - 1, c1, c2, c3)
        take = (wp < wid) & (segl == s0)
        mv = jnp.full((LANES,), jnp.where(take, 1.0, 0.0), jnp.float32)
        cden = cden + mv * td_v[pl.ds(wp * LANES, LANES)]
        for k in range(KD):
            cnum[k] = cnum[k] + mv * tn_v[pl.ds(wp * D + LANES * k, LANES)]

    def body(r, carry):
        den = carry[0]
        nums = carry[1:]
        off = r * D
        cks = [ctx_v[pl.ds(off + LANES * k, LANES)] for k in range(KD)]
        acc = cks[0] * th[0]
        for k in range(1, KD):
            acc = acc + cks[k] * th[k]
        e = jnp.exp(jnp.full((LANES,), jnp.sum(acc), jnp.float32))
        rg = row0 + r
        is_start = (rg == c1) | (rg == c2) | (rg == c3)
        kv = jnp.full((LANES,), jnp.where(is_start, 0.0, 1.0), jnp.float32)
        den = den * kv + e
        nums = tuple(n * kv + e * ck for n, ck in zip(nums, cks))
        for k in range(KD):
            out_v[pl.ds(off + LANES * k, LANES)] = nums[k] / den
        return (den,) + nums

    lax.fori_loop(0, CHUNK, body, (cden,) + tuple(cnum))
    pltpu.sync_copy(out_v, out_hbm.at[pl.ds(wid * CHUNK_E, CHUNK_E)])


_phase1 = pl.kernel(
    _phase1_body,
    out_type=(jax.ShapeDtypeStruct((NW * D,), jnp.float32),
              jax.ShapeDtypeStruct((NW * LANES,), jnp.float32)),
    mesh=_mesh,
    compiler_params=_cparams,
    scratch_types=[
        pltpu.VMEM((CHUNK_E,), jnp.float32),
        pltpu.VMEM((D,), jnp.float32),
        pltpu.VMEM((LANES,), jnp.int32),
        pltpu.VMEM((D,), jnp.float32),
        pltpu.VMEM((LANES,), jnp.float32),
    ],
)

_phase2 = pl.kernel(
    _phase2_body,
    out_type=jax.ShapeDtypeStruct((T * D,), jnp.float32),
    mesh=_mesh,
    compiler_params=_cparams,
    scratch_types=[
        pltpu.VMEM((CHUNK_E,), jnp.float32),
        pltpu.VMEM((CHUNK_E,), jnp.float32),
        pltpu.VMEM((D,), jnp.float32),
        pltpu.VMEM((LANES,), jnp.int32),
        pltpu.VMEM((NW * D,), jnp.float32),
        pltpu.VMEM((NW * LANES,), jnp.float32),
    ],
)


@jax.jit
def kernel(context, cu_seqlens, context_theta):
    ctx_flat = context.reshape(-1)
    th_flat = context_theta.reshape(-1)
    cu_pad = jnp.concatenate(
        [cu_seqlens.astype(jnp.int32),
         jnp.zeros((LANES - cu_seqlens.shape[0],), jnp.int32)])
    tnum, tden = _phase1(ctx_flat, cu_pad, th_flat)
    out_flat = _phase2(ctx_flat, cu_pad, th_flat, tnum, tden)
    return out_flat.reshape(T, D)
